# Initial kernel scaffold; baseline (speedup 1.0000x reference)
import jax
import jax.numpy as jnp
from jax.experimental import pallas as pl


def kernel(pred, target):
    raise NotImplementedError("write your pallas kernel here")


# speedup vs baseline: 13.6196x; 13.6196x over previous
import functools
import jax
import jax.numpy as jnp
from jax import lax
from jax.experimental import pallas as pl

_OHEM_RATIO = 0.25
_EPS = 1e-07


def _body(pred_ref, target_ref, out_ref):
    pred = pred_ref[:]
    target = target_ref[:]
    loss = jnp.maximum(pred, 0.0) - pred * target + jnp.log1p(jnp.exp(-jnp.abs(pred)))
    bits = lax.bitcast_convert_type(loss, jnp.int32)
    n = pred.size
    k = int(n * _OHEM_RATIO)

    def step(i, t):
        cand = t | lax.shift_left(1, 30 - i)
        cnt = jnp.sum((bits >= cand).astype(jnp.int32))
        return lax.select(cnt >= k, cand, t)

    t = lax.fori_loop(0, 31, step, jnp.int32(0))
    mask = (bits >= t).astype(jnp.float32)
    s = jnp.sum(loss * mask)
    c = jnp.sum(mask)
    out_ref[:, :] = jnp.reshape(s / (c + _EPS), (1, 1))


@jax.jit
def kernel(pred, target):
    out = pl.pallas_call(
        _body,
        out_shape=jax.ShapeDtypeStruct((1, 1), jnp.float32),
    )(pred, target)
    return out[0, 0]
